# Initial kernel scaffold; baseline (speedup 1.0000x reference)
import jax
import jax.numpy as jnp
from jax.experimental import pallas as pl


def kernel(feature, edge_index, preference, W, b):
    raise NotImplementedError("write your pallas kernel here")


# speedup vs baseline: 19.6537x; 19.6537x over previous
import functools

import jax
import jax.numpy as jnp
from jax import lax
from jax.experimental import pallas as pl
from jax.experimental.pallas import tpu as pltpu
from jax.experimental.pallas import tpu_sc as plsc

_N = 50000
_NP = 50048
_D = 32
_NC = 2
_NS = 16
_NW = _NC * _NS
_L = 16
_B = 128
_IC = 512
_NOUT_U = 25088
_EPS = 1e-16

_SC_PARAMS = dict(needs_layout_passes=False, use_tc_tiling_on_sc=False)


def _conv_body(ept, nout, ndir, need_ex, epb, *refs):
    if need_ex:
        (x_ref, src_ref, dst_ref, numer_ref, denom_ref, ex_ref,
         acc_n, accd, sidx, didx, xs, xd, m, exb, zbuf, sem_s, sem_d) = refs
    else:
        (x_ref, src_ref, dst_ref, numer_ref, denom_ref,
         acc_n, accd, sidx, didx, xs, xd, m, exb, zbuf, sem_s, sem_d) = refs
        ex_ref = None

    c = lax.axis_index("c")
    s = lax.axis_index("s")
    tid = c * _NS + s

    zero16 = jnp.zeros((_L,), jnp.float32)

    def _zz(i, _):
        zbuf[pl.ds(i * _L, _L)] = zero16
        return 0
    lax.fori_loop(0, 3200 // _L, _zz, 0)

    def _zm(i, _):
        m[i, pl.ds(0, _L)] = zero16
        m[i, pl.ds(_L, _L)] = zero16
        return 0
    lax.fori_loop(0, _B, _zm, 0)

    r0 = s * (_N // _NS)
    def _zs(i, _):
        pltpu.sync_copy(m.at[pl.ds(0, 125)], acc_n.at[pl.ds(r0 + i * 125, 125)])
        return 0
    lax.fori_loop(0, (_N // _NS) // 125, _zs, 0)
    pltpu.sync_copy(zbuf.at[pl.ds(0, _NP // _NS)],
                    accd.at[pl.ds(s * (_NP // _NS), _NP // _NS)])
    plsc.subcore_barrier()

    iota16 = lax.iota(jnp.int32, _L)
    rows_per_chunk = _IC // _B

    for p in range(ndir):
        s_src = src_ref if p == 0 else dst_ref
        s_dst = dst_ref if p == 0 else src_ref

        def _chunk(ib, _, _p=p, _ss=s_src, _sd=s_dst):
            base_row = tid * (ept // _B) + ib * rows_per_chunk
            pltpu.sync_copy(_ss.at[pl.ds(base_row, rows_per_chunk)], sidx)
            pltpu.sync_copy(_sd.at[pl.ds(base_row, rows_per_chunk)], didx)
            for jb in range(rows_per_chunk):
                pltpu.async_copy(x_ref.at[sidx.at[jb]], xs, sem_s).wait()
                pltpu.async_copy(x_ref.at[didx.at[jb]], xd, sem_d).wait()

                def _group(g, _, _jb=jb):
                    off = g * _L
                    vs = sidx[_jb, pl.ds(off, _L)]
                    vd = didx[_jb, pl.ds(off, _L)]
                    vdot = jnp.zeros((_L,), jnp.float32)
                    for i in range(_L):
                        e = off + i
                        s0 = xs[e, pl.ds(0, _L)]
                        s1 = xs[e, pl.ds(_L, _L)]
                        d0 = xd[e, pl.ds(0, _L)]
                        d1 = xd[e, pl.ds(_L, _L)]
                        di = jnp.sum(s0 * d0 + s1 * d1)
                        vdot = jnp.where(iota16 == i, di, vdot)
                    ex = jnp.where(vs != vd, jnp.exp(vdot), 0.0)
                    exb[_jb, pl.ds(off, _L)] = ex
                    for i in range(_L):
                        e = off + i
                        exi = ex[i]
                        m[e, pl.ds(0, _L)] = exi * xs[e, pl.ds(0, _L)]
                        m[e, pl.ds(_L, _L)] = exi * xs[e, pl.ds(_L, _L)]
                    return 0

                lax.fori_loop(0, _B // _L, _group, 0)
                pltpu.sync_copy(m, acc_n.at[didx.at[jb]], add=True)
                pltpu.sync_copy(exb.at[jb], accd.at[didx.at[jb]], add=True)
            if need_ex:
                pltpu.sync_copy(
                    exb, ex_ref.at[pl.ds(_p * epb + base_row, rows_per_chunk)])
            return 0

        lax.fori_loop(0, ept // _IC, _chunk, 0)

    plsc.subcore_barrier()

    dstripe = _NP // _NS
    pltpu.sync_copy(accd.at[pl.ds(s * dstripe, dstripe)],
                    denom_ref.at[c, pl.ds(s * dstripe, dstripe)])

    rows_per_tile = nout // _NS
    chunk = 112 if rows_per_tile % 125 else 125
    q0 = s * rows_per_tile
    def _cp(i, _):
        pltpu.sync_copy(acc_n.at[pl.ds(q0 + i * chunk, chunk)],
                        m.at[pl.ds(0, chunk)])
        pltpu.sync_copy(m.at[pl.ds(0, chunk)],
                        numer_ref.at[c, pl.ds(q0 + i * chunk, chunk)])
        return 0
    lax.fori_loop(0, rows_per_tile // chunk, _cp, 0)


def _make_conv(ept, nout, ndir, need_ex, epb):
    mesh = plsc.VectorSubcoreMesh(core_axis_name="c", subcore_axis_name="s",
                                  num_cores=_NC, num_subcores=_NS)
    outs = [
        jax.ShapeDtypeStruct((_NC, nout, _D), jnp.float32),
        jax.ShapeDtypeStruct((_NC, _NP), jnp.float32),
    ]
    if need_ex:
        outs.append(jax.ShapeDtypeStruct((ndir * epb, _B), jnp.float32))
    scratch = [
        pltpu.VMEM_SHARED((_N, _D), jnp.float32),
        pltpu.VMEM_SHARED((_NP,), jnp.float32),
        pltpu.VMEM((_IC // _B, _B), jnp.int32),
        pltpu.VMEM((_IC // _B, _B), jnp.int32),
        pltpu.VMEM((_B, _D), jnp.float32),
        pltpu.VMEM((_B, _D), jnp.float32),
        pltpu.VMEM((_B, _D), jnp.float32),
        pltpu.VMEM((_IC // _B, _B), jnp.float32),
        pltpu.VMEM((3200,), jnp.float32),
        pltpu.SemaphoreType.DMA,
        pltpu.SemaphoreType.DMA,
    ]
    body = functools.partial(_conv_body, ept, nout, ndir, need_ex, epb)
    return pl.kernel(body, out_type=tuple(outs), mesh=mesh,
                     compiler_params=pltpu.CompilerParams(**_SC_PARAMS),
                     scratch_types=scratch)


def _alpha_body(ept, epb, den_ref, src_ref, dst_ref, ex_ref, al_ref,
                denl, didx, exb, alb):
    c = lax.axis_index("c")
    s = lax.axis_index("s")
    tid = c * _NS + s
    pltpu.sync_copy(den_ref, denl)
    rows_per_chunk = _IC // _B
    for p in range(2):
        dref = dst_ref if p == 0 else src_ref

        def _chunk(ib, _, _p=p, _dref=dref):
            base = tid * (ept // _B) + ib * rows_per_chunk
            pltpu.sync_copy(_dref.at[pl.ds(base, rows_per_chunk)], didx)
            pltpu.sync_copy(ex_ref.at[pl.ds(_p * epb + base, rows_per_chunk)],
                            exb)
            for jb in range(rows_per_chunk):
                def _grp(g, _, _jb=jb):
                    off = g * _L
                    vd = didx[_jb, pl.ds(off, _L)]
                    dv = plsc.load_gather(denl, [vd])
                    ev = exb[_jb, pl.ds(off, _L)]
                    alb[_jb, pl.ds(off, _L)] = ev / (dv + _EPS)
                    return 0
                lax.fori_loop(0, _B // _L, _grp, 0)
            pltpu.sync_copy(alb, al_ref.at[pl.ds(_p * epb + base,
                                                 rows_per_chunk)])
            return 0

        lax.fori_loop(0, ept // _IC, _chunk, 0)


def _make_alpha(ept, epb):
    mesh = plsc.VectorSubcoreMesh(core_axis_name="c", subcore_axis_name="s",
                                  num_cores=_NC, num_subcores=_NS)
    scratch = [
        pltpu.VMEM((_N,), jnp.float32),
        pltpu.VMEM((_IC // _B, _B), jnp.int32),
        pltpu.VMEM((_IC // _B, _B), jnp.float32),
        pltpu.VMEM((_IC // _B, _B), jnp.float32),
    ]
    body = functools.partial(_alpha_body, ept, epb)
    return pl.kernel(body,
                     out_type=jax.ShapeDtypeStruct((2 * epb, _B), jnp.float32),
                     mesh=mesh,
                     compiler_params=pltpu.CompilerParams(**_SC_PARAMS),
                     scratch_types=scratch)


_RB = 1000


def _prep_feats_body(f_ref, w_ref, b_ref, o_ref):
    y = lax.dot_general(f_ref[...], w_ref[...],
                        (((1,), (1,)), ((), ())),
                        preferred_element_type=jnp.float32)
    y = y + b_ref[...]
    y = jnp.where(y > 0, y, 0.01 * y)
    n = jnp.sqrt(jnp.sum(y * y, axis=1, keepdims=True))
    o_ref[...] = y / jnp.maximum(n, 1e-12)


def _norm_body(p_ref, o_ref):
    p = p_ref[...]
    n = jnp.sqrt(jnp.sum(p * p, axis=1, keepdims=True))
    o_ref[...] = p / jnp.maximum(n, 1e-12)


def _route_body(p_ref, n_ref, d_ref, o_ref):
    num = n_ref[0] + n_ref[1]
    den = d_ref[0, 0, 0, :] + d_ref[1, 0, 0, :]
    out = num / (den[:, None] + _EPS)
    p2 = p_ref[...] + out
    nn = jnp.sqrt(jnp.sum(p2 * p2, axis=1, keepdims=True))
    o_ref[...] = p2 / jnp.maximum(nn, 1e-12)


def _final_body(x_ref, n_ref, d_ref, y_ref, dt_ref):
    num = n_ref[0] + n_ref[1]
    den = d_ref[0, 0, 0, :] + d_ref[1, 0, 0, :]
    out = num / (den[:, None] + _EPS)
    out = jnp.where(out > 0, out, 0.01 * out)
    y_ref[...] = x_ref[...] + out
    dt_ref[...] = jnp.broadcast_to(den[None, None, None, :],
                                   (8, 1, 1, den.shape[0]))


def kernel(feature, edge_index, preference, W, b):
    nu = preference.shape[0]
    ni = feature.shape[0]
    n_nodes = nu + ni
    e = edge_index.shape[1]

    ep = -(-e // (_NW * _IC)) * (_NW * _IC)
    epb = ep // _B
    ept = ep // _NW
    pad = ep - e
    src = jnp.concatenate([edge_index[0], jnp.zeros((pad,), jnp.int32)])
    dst = jnp.concatenate([edge_index[1], jnp.zeros((pad,), jnp.int32)])
    src2d = src.reshape(epb, _B)
    dst2d = dst.reshape(epb, _B)

    feats = pl.pallas_call(
        _prep_feats_body,
        grid=(ni // _RB,),
        in_specs=[
            pl.BlockSpec((_RB, feature.shape[1]), lambda i: (i, 0)),
            pl.BlockSpec(W.shape, lambda i: (0, 0)),
            pl.BlockSpec((1, _D), lambda i: (0, 0)),
        ],
        out_specs=pl.BlockSpec((_RB, _D), lambda i: (i, 0)),
        out_shape=jax.ShapeDtypeStruct((ni, _D), jnp.float32),
    )(feature, W, b.reshape(1, _D))

    pref = pl.pallas_call(
        _norm_body,
        grid=(nu // _RB,),
        in_specs=[pl.BlockSpec((_RB, _D), lambda i: (i, 0))],
        out_specs=pl.BlockSpec((_RB, _D), lambda i: (i, 0)),
        out_shape=jax.ShapeDtypeStruct((nu, _D), jnp.float32),
    )(preference)

    conv_route = _make_conv(ept, _NOUT_U, 1, False, epb)
    route = pl.pallas_call(
        _route_body,
        grid=(nu // _RB,),
        in_specs=[
            pl.BlockSpec((_RB, _D), lambda i: (i, 0)),
            pl.BlockSpec((_NC, _RB, _D), lambda i: (0, i, 0)),
            pl.BlockSpec((_NC, 1, 1, _RB), lambda i: (0, i, 0, 0)),
        ],
        out_specs=pl.BlockSpec((_RB, _D), lambda i: (i, 0)),
        out_shape=jax.ShapeDtypeStruct((nu, _D), jnp.float32),
    )

    for _ in range(2):
        x = jnp.concatenate([pref, feats], axis=0)
        numer_p, denom_p = conv_route(x, src2d, dst2d)
        d3 = denom_p[:, :nu].reshape(_NC, nu // _RB, 1, _RB)
        pref = route(pref, numer_p[:, :nu], d3)

    x = jnp.concatenate([pref, feats], axis=0)
    conv_final = _make_conv(ept, n_nodes, 2, True, epb)
    numer_p, denom_p, ex2d = conv_final(x, src2d, dst2d)

    y, dt = pl.pallas_call(
        _final_body,
        grid=(n_nodes // _RB,),
        in_specs=[
            pl.BlockSpec((_RB, _D), lambda i: (i, 0)),
            pl.BlockSpec((_NC, _RB, _D), lambda i: (0, i, 0)),
            pl.BlockSpec((_NC, 1, 1, _RB), lambda i: (0, i, 0, 0)),
        ],
        out_specs=[
            pl.BlockSpec((_RB, _D), lambda i: (i, 0)),
            pl.BlockSpec((8, 1, 1, _RB), lambda i: (0, i, 0, 0)),
        ],
        out_shape=[
            jax.ShapeDtypeStruct((n_nodes, _D), jnp.float32),
            jax.ShapeDtypeStruct((8, n_nodes // _RB, 1, _RB), jnp.float32),
        ],
    )(x, numer_p, denom_p[:, :n_nodes].reshape(_NC, n_nodes // _RB, 1, _RB))

    al2d = _make_alpha(ept, epb)(dt[0].reshape(n_nodes), src2d, dst2d, ex2d)
    al = al2d.reshape(-1)
    alpha = jnp.concatenate([al[:e], al[ep:ep + e]])[:, None]
    return (y, alpha)
